# baseline (speedup 1.0000x reference)
import functools

import jax
import jax.numpy as jnp
from jax import lax
from jax.experimental import pallas as pl
from jax.experimental.pallas import tpu as pltpu
from jax.experimental.pallas import tpu_sc as plsc

_N, _D, _E, _R, _M, _G = 2048, 1024, 64, 16, 8, 8
_TILE = 1024
_NEG = -1e30

_NC, _NS, _L = 2, 16, 16
_NW = _NC * _NS
_TPW = _N // _NW


def _score_body(h_ref, wcat_ref, bias_ref, s_ref):
    s = jnp.dot(h_ref[...], wcat_ref[...], preferred_element_type=jnp.float32)
    s_ref[...] = s + bias_ref[0:1, :]


def _sc_route_body(scores_hbm, eid_hbm, gate_hbm, gidx_hbm,
                   sc_v, eid_v, gate_v, gidx_v):
    wid = lax.axis_index("s") * _NC + lax.axis_index("c")
    base = wid * _TPW
    pltpu.sync_copy(scores_hbm.at[pl.ds(base * 128, _TPW * 128)], sc_v)
    lanes = lax.iota(jnp.int32, _L)
    zero = jnp.zeros((_L,), jnp.int32)
    for b in range(_TPW // _L):
        toks = lanes + (b * _L)
        rowoff = toks * 128
        bv = jnp.full((_L,), _NEG, jnp.float32)
        bi = zero
        for j in range(_G):
            v = plsc.load_gather(sc_v, [rowoff + j])
            upd = v > bv
            bv = jnp.where(upd, v, bv)
            bi = jnp.where(upd, zero + j, bi)
        v1 = jnp.full((_L,), _NEG, jnp.float32)
        v2 = jnp.full((_L,), _NEG, jnp.float32)
        i1 = zero
        i2 = zero
        for m in range(_M):
            v = plsc.load_gather(sc_v, [rowoff + (bi * _M + (_G + m))])
            gt1 = v > v1
            gt2 = v > v2
            mv = zero + m
            nv2 = jnp.where(gt1, v1, jnp.where(gt2, v, v2))
            ni2 = jnp.where(gt1, i1, jnp.where(gt2, mv, i2))
            v1 = jnp.where(gt1, v, v1)
            i1 = jnp.where(gt1, mv, i1)
            v2 = nv2
            i2 = ni2
        t = jnp.exp(v2 - v1)
        g2 = t / (1.0 + t)
        g1 = 1.0 - g2
        plsc.store_scatter(eid_v, [toks * 2], bi * _M + i1)
        plsc.store_scatter(eid_v, [toks * 2 + 1], bi * _M + i2)
        plsc.store_scatter(gate_v, [toks * 2], g1)
        plsc.store_scatter(gate_v, [toks * 2 + 1], g2)
        gidx_v[pl.ds(b * _L, _L)] = bi
    pltpu.sync_copy(eid_v, eid_hbm.at[pl.ds(base * 2, _TPW * 2)])
    pltpu.sync_copy(gate_v, gate_hbm.at[pl.ds(base * 2, _TPW * 2)])
    pltpu.sync_copy(gidx_v, gidx_hbm.at[pl.ds(base, _TPW)])


def _expert_body(h_ref, w1_ref, w2_ref, eid_ref, gate_ref, out_ref):
    tmp = jnp.dot(h_ref[...], w1_ref[...], preferred_element_type=jnp.float32)
    tmp = jnp.maximum(tmp, 0.0)
    e1 = eid_ref[:, 0:1]
    e2 = eid_ref[:, 1:2]
    g1 = gate_ref[:, 0:1]
    g2 = gate_ref[:, 1:2]
    elane = lax.broadcasted_iota(jnp.int32, tmp.shape, 1) // _R
    wexp = jnp.where(elane == e1, g1, 0.0) + jnp.where(elane == e2, g2, 0.0)
    out_ref[...] = jnp.dot(tmp * wexp, w2_ref[...],
                           preferred_element_type=jnp.float32)


def kernel(h, k, Wg, bg, local_router, W1, W2):
    f32 = jnp.float32
    wcat = jnp.zeros((_D, 128), f32)
    wcat = wcat.at[:, :_G].set(Wg.T)
    wcat = wcat.at[:, _G:_G + _G * _M].set(
        local_router.transpose(1, 0, 2).reshape(_D, _G * _M))
    bias = jnp.zeros((8, 128), f32).at[0, :_G].set(bg)
    w1t = W1.transpose(1, 0, 2).reshape(_D, _E * _R)
    w2f = W2.reshape(_E * _R, _D)

    grid = _N // _TILE
    scores = pl.pallas_call(
        _score_body,
        grid=(grid,),
        in_specs=[
            pl.BlockSpec((_TILE, _D), lambda i: (i, 0)),
            pl.BlockSpec((_D, 128), lambda i: (0, 0)),
            pl.BlockSpec((8, 128), lambda i: (0, 0)),
        ],
        out_specs=pl.BlockSpec((_TILE, 128), lambda i: (i, 0)),
        out_shape=jax.ShapeDtypeStruct((_N, 128), f32),
    )(h, wcat, bias)

    sc_route = pl.kernel(
        _sc_route_body,
        out_type=[
            jax.ShapeDtypeStruct((_N * 2,), jnp.int32),
            jax.ShapeDtypeStruct((_N * 2,), f32),
            jax.ShapeDtypeStruct((_N,), jnp.int32),
        ],
        mesh=plsc.VectorSubcoreMesh(core_axis_name="c", subcore_axis_name="s",
                                    num_cores=_NC, num_subcores=_NS),
        scratch_types=[
            pltpu.VMEM((_TPW * 128,), f32),
            pltpu.VMEM((_TPW * 2,), jnp.int32),
            pltpu.VMEM((_TPW * 2,), f32),
            pltpu.VMEM((_TPW,), jnp.int32),
        ],
        compiler_params=pltpu.CompilerParams(needs_layout_passes=False),
    )
    eid_flat, gate_flat, gidx = sc_route(scores.reshape(_N * 128))
    eid = eid_flat.reshape(_N, 2)
    gate = gate_flat.reshape(_N, 2)

    out = pl.pallas_call(
        _expert_body,
        grid=(grid,),
        in_specs=[
            pl.BlockSpec((_TILE, _D), lambda i: (i, 0)),
            pl.BlockSpec((_D, _E * _R), lambda i: (0, 0)),
            pl.BlockSpec((_E * _R, _D), lambda i: (0, 0)),
            pl.BlockSpec((_TILE, 2), lambda i: (i, 0)),
            pl.BlockSpec((_TILE, 2), lambda i: (i, 0)),
        ],
        out_specs=pl.BlockSpec((_TILE, _D), lambda i: (i, 0)),
        out_shape=jax.ShapeDtypeStruct((_N, _D), f32),
    )(h, w1t, w2f, eid, gate)

    gate = gate + (jnp.asarray(k, gate.dtype) - 2.0)
    return out, eid, gate, gidx


# speedup vs baseline: 1.0003x; 1.0003x over previous
import jax
import jax.numpy as jnp
from jax import lax
from jax.experimental import pallas as pl
from jax.experimental.pallas import tpu as pltpu
from jax.experimental.pallas import tpu_sc as plsc

_N, _D, _E, _R, _M, _G = 2048, 1024, 64, 16, 8, 8
_TILE = 512
_NEG = -1e30

_NC, _NS, _L = 2, 16, 16
_NW = _NC * _NS
_TPW = _N // _NW


def _score_body(h_ref, wcat_ref, bias_ref, s_ref):
    s = jnp.dot(h_ref[...], wcat_ref[...], preferred_element_type=jnp.float32)
    s_ref[...] = s + bias_ref[0:1, :]


def _sc_route_body(scores_hbm, eid_hbm, gate_hbm, gidx_hbm,
                   sc_v, eid_v, gate_v, gidx_v):
    wid = lax.axis_index("s") * _NC + lax.axis_index("c")
    base = wid * _TPW
    pltpu.sync_copy(scores_hbm.at[pl.ds(base * 128, _TPW * 128)], sc_v)
    lanes = lax.iota(jnp.int32, _L)
    zero = jnp.zeros((_L,), jnp.int32)
    for b in range(_TPW // _L):
        toks = lanes + (b * _L)
        rowoff = toks * 128
        bv = jnp.full((_L,), _NEG, jnp.float32)
        bi = zero
        for j in range(_G):
            v = plsc.load_gather(sc_v, [rowoff + j])
            upd = v > bv
            bv = jnp.where(upd, v, bv)
            bi = jnp.where(upd, zero + j, bi)
        v1 = jnp.full((_L,), _NEG, jnp.float32)
        v2 = jnp.full((_L,), _NEG, jnp.float32)
        i1 = zero
        i2 = zero
        for m in range(_M):
            v = plsc.load_gather(sc_v, [rowoff + (bi * _M + (_G + m))])
            gt1 = v > v1
            gt2 = v > v2
            mv = zero + m
            nv2 = jnp.where(gt1, v1, jnp.where(gt2, v, v2))
            ni2 = jnp.where(gt1, i1, jnp.where(gt2, mv, i2))
            v1 = jnp.where(gt1, v, v1)
            i1 = jnp.where(gt1, mv, i1)
            v2 = nv2
            i2 = ni2
        t = jnp.exp(v2 - v1)
        g2 = t / (1.0 + t)
        g1 = 1.0 - g2
        plsc.store_scatter(eid_v, [toks * 2], bi * _M + i1)
        plsc.store_scatter(eid_v, [toks * 2 + 1], bi * _M + i2)
        plsc.store_scatter(gate_v, [toks * 2], g1)
        plsc.store_scatter(gate_v, [toks * 2 + 1], g2)
        gidx_v[pl.ds(b * _L, _L)] = bi
    pltpu.sync_copy(eid_v, eid_hbm.at[pl.ds(base * 2, _TPW * 2)])
    pltpu.sync_copy(gate_v, gate_hbm.at[pl.ds(base * 2, _TPW * 2)])
    pltpu.sync_copy(gidx_v, gidx_hbm.at[pl.ds(base, _TPW)])


def _expert_body(h_ref, w1_ref, w2_ref, eid_ref, gate_ref, out_ref):
    tmp = jnp.dot(h_ref[...], w1_ref[...], preferred_element_type=jnp.float32)
    tmp = jnp.maximum(tmp, 0.0)
    e1 = eid_ref[:, 0:1]
    e2 = eid_ref[:, 1:2]
    g1 = gate_ref[:, 0:1]
    g2 = gate_ref[:, 1:2]
    elane = lax.broadcasted_iota(jnp.int32, tmp.shape, 1) // _R
    wexp = jnp.where(elane == e1, g1, 0.0) + jnp.where(elane == e2, g2, 0.0)
    out_ref[...] = jnp.dot(tmp * wexp, w2_ref[...],
                           preferred_element_type=jnp.float32)


def kernel(h, k, Wg, bg, local_router, W1, W2):
    f32 = jnp.float32
    wcat = jnp.zeros((_D, 128), f32)
    wcat = wcat.at[:, :_G].set(Wg.T)
    wcat = wcat.at[:, _G:_G + _G * _M].set(
        local_router.transpose(1, 0, 2).reshape(_D, _G * _M))
    bias = jnp.zeros((8, 128), f32).at[0, :_G].set(bg)
    w1t = W1.transpose(1, 0, 2).reshape(_D, _E * _R)
    w2f = W2.reshape(_E * _R, _D)

    grid = _N // _TILE
    scores = pl.pallas_call(
        _score_body,
        grid=(grid,),
        in_specs=[
            pl.BlockSpec((_TILE, _D), lambda i: (i, 0)),
            pl.BlockSpec((_D, 128), lambda i: (0, 0)),
            pl.BlockSpec((8, 128), lambda i: (0, 0)),
        ],
        out_specs=pl.BlockSpec((_TILE, 128), lambda i: (i, 0)),
        out_shape=jax.ShapeDtypeStruct((_N, 128), f32),
    )(h, wcat, bias)

    sc_route = pl.kernel(
        _sc_route_body,
        out_type=[
            jax.ShapeDtypeStruct((_N * 2,), jnp.int32),
            jax.ShapeDtypeStruct((_N * 2,), f32),
            jax.ShapeDtypeStruct((_N,), jnp.int32),
        ],
        mesh=plsc.VectorSubcoreMesh(core_axis_name="c", subcore_axis_name="s",
                                    num_cores=_NC, num_subcores=_NS),
        scratch_types=[
            pltpu.VMEM((_TPW * 128,), f32),
            pltpu.VMEM((_TPW * 2,), jnp.int32),
            pltpu.VMEM((_TPW * 2,), f32),
            pltpu.VMEM((_TPW,), jnp.int32),
        ],
        compiler_params=pltpu.CompilerParams(needs_layout_passes=False),
    )
    eid_flat, gate_flat, gidx = sc_route(scores.reshape(_N * 128))
    eid = eid_flat.reshape(_N, 2)
    gate = gate_flat.reshape(_N, 2)

    out = pl.pallas_call(
        _expert_body,
        grid=(grid,),
        in_specs=[
            pl.BlockSpec((_TILE, _D), lambda i: (i, 0)),
            pl.BlockSpec((_D, _E * _R), lambda i: (0, 0)),
            pl.BlockSpec((_E * _R, _D), lambda i: (0, 0)),
            pl.BlockSpec((_TILE, 2), lambda i: (i, 0)),
            pl.BlockSpec((_TILE, 2), lambda i: (i, 0)),
        ],
        out_specs=pl.BlockSpec((_TILE, _D), lambda i: (i, 0)),
        out_shape=jax.ShapeDtypeStruct((_N, _D), f32),
    )(h, w1t, w2f, eid, gate)

    gate = gate + (jnp.asarray(k, gate.dtype) - 2.0)
    return out, eid, gate, gidx


# speedup vs baseline: 1.0011x; 1.0008x over previous
import jax
import jax.numpy as jnp
from jax import lax
from jax.experimental import pallas as pl
from jax.experimental.pallas import tpu as pltpu
from jax.experimental.pallas import tpu_sc as plsc

_N, _D, _E, _R, _M, _G = 2048, 1024, 64, 16, 8, 8
_TILE = 512
_NEG = -1e30

_NC, _NS, _L = 2, 16, 16
_NW = _NC * _NS
_TPW = _N // _NW


def _score_body(h_ref, wcat_ref, bias_ref, s_ref):
    s = jnp.dot(h_ref[...], wcat_ref[...], preferred_element_type=jnp.float32)
    s_ref[...] = s + bias_ref[0:1, :]


def _sc_route_body(scores_hbm, eid_hbm, gate_hbm, gidx_hbm,
                   sc_v, eid_v, gate_v, gidx_v):
    wid = lax.axis_index("s") * _NC + lax.axis_index("c")
    base = wid * _TPW
    pltpu.sync_copy(scores_hbm.at[pl.ds(base * 128, _TPW * 128)], sc_v)
    lanes = lax.iota(jnp.int32, _L)
    zero = jnp.zeros((_L,), jnp.int32)
    for b in range(_TPW // _L):
        toks = lanes + (b * _L)
        rowoff = toks * 128
        bv = jnp.full((_L,), _NEG, jnp.float32)
        bi = zero
        for j in range(_G):
            v = plsc.load_gather(sc_v, [rowoff + j])
            upd = v > bv
            bv = jnp.where(upd, v, bv)
            bi = jnp.where(upd, zero + j, bi)
        v1 = jnp.full((_L,), _NEG, jnp.float32)
        v2 = jnp.full((_L,), _NEG, jnp.float32)
        i1 = zero
        i2 = zero
        for m in range(_M):
            v = plsc.load_gather(sc_v, [rowoff + (bi * _M + (_G + m))])
            gt1 = v > v1
            gt2 = v > v2
            mv = zero + m
            nv2 = jnp.where(gt1, v1, jnp.where(gt2, v, v2))
            ni2 = jnp.where(gt1, i1, jnp.where(gt2, mv, i2))
            v1 = jnp.where(gt1, v, v1)
            i1 = jnp.where(gt1, mv, i1)
            v2 = nv2
            i2 = ni2
        t = jnp.exp(v2 - v1)
        g2 = t / (1.0 + t)
        g1 = 1.0 - g2
        plsc.store_scatter(eid_v, [toks * 2], bi * _M + i1)
        plsc.store_scatter(eid_v, [toks * 2 + 1], bi * _M + i2)
        plsc.store_scatter(gate_v, [toks * 2], g1)
        plsc.store_scatter(gate_v, [toks * 2 + 1], g2)
        gidx_v[pl.ds(b * _L, _L)] = bi
    pltpu.sync_copy(eid_v, eid_hbm.at[pl.ds(base * 2, _TPW * 2)])
    pltpu.sync_copy(gate_v, gate_hbm.at[pl.ds(base * 2, _TPW * 2)])
    pltpu.sync_copy(gidx_v, gidx_hbm.at[pl.ds(base, _TPW)])


def _expert_body(h_ref, w1_ref, w2_ref, eid_ref, gate_ref, out_ref):
    tmp = jnp.dot(h_ref[...], w1_ref[...], preferred_element_type=jnp.float32)
    tmp = jnp.maximum(tmp, 0.0)
    e1 = eid_ref[:, 0:1]
    e2 = eid_ref[:, 1:2]
    g1 = gate_ref[:, 0:1]
    g2 = gate_ref[:, 1:2]
    elane = lax.broadcasted_iota(jnp.int32, tmp.shape, 1) // _R
    wexp = jnp.where(elane == e1, g1, 0.0) + jnp.where(elane == e2, g2, 0.0)
    out_ref[...] = jnp.dot(tmp * wexp, w2_ref[...],
                           preferred_element_type=jnp.float32)


def kernel(h, k, Wg, bg, local_router, W1, W2):
    f32 = jnp.float32
    wcat = jnp.zeros((_D, 128), f32)
    wcat = wcat.at[:, :_G].set(Wg.T)
    wcat = wcat.at[:, _G:_G + _G * _M].set(
        local_router.transpose(1, 0, 2).reshape(_D, _G * _M))
    bias = jnp.zeros((8, 128), f32).at[0, :_G].set(bg)
    w1t = W1.transpose(1, 0, 2).reshape(_D, _E * _R)
    w2f = W2.reshape(_E * _R, _D)

    grid = _N // _TILE
    scores = pl.pallas_call(
        _score_body,
        grid=(grid,),
        in_specs=[
            pl.BlockSpec((_TILE, _D), lambda i: (i, 0)),
            pl.BlockSpec((_D, 128), lambda i: (0, 0)),
            pl.BlockSpec((8, 128), lambda i: (0, 0)),
        ],
        out_specs=pl.BlockSpec((_TILE, 128), lambda i: (i, 0)),
        out_shape=jax.ShapeDtypeStruct((_N, 128), f32),
        compiler_params=pltpu.CompilerParams(allow_input_fusion=[False, True, True]),
    )(h, wcat, bias)

    sc_route = pl.kernel(
        _sc_route_body,
        out_type=[
            jax.ShapeDtypeStruct((_N * 2,), jnp.int32),
            jax.ShapeDtypeStruct((_N * 2,), f32),
            jax.ShapeDtypeStruct((_N,), jnp.int32),
        ],
        mesh=plsc.VectorSubcoreMesh(core_axis_name="c", subcore_axis_name="s",
                                    num_cores=_NC, num_subcores=_NS),
        scratch_types=[
            pltpu.VMEM((_TPW * 128,), f32),
            pltpu.VMEM((_TPW * 2,), jnp.int32),
            pltpu.VMEM((_TPW * 2,), f32),
            pltpu.VMEM((_TPW,), jnp.int32),
        ],
        compiler_params=pltpu.CompilerParams(needs_layout_passes=False),
    )
    eid_flat, gate_flat, gidx = sc_route(scores.reshape(_N * 128))
    eid = eid_flat.reshape(_N, 2)
    gate = gate_flat.reshape(_N, 2)

    out = pl.pallas_call(
        _expert_body,
        grid=(grid,),
        in_specs=[
            pl.BlockSpec((_TILE, _D), lambda i: (i, 0)),
            pl.BlockSpec((_D, _E * _R), lambda i: (0, 0)),
            pl.BlockSpec((_E * _R, _D), lambda i: (0, 0)),
            pl.BlockSpec((_TILE, 2), lambda i: (i, 0)),
            pl.BlockSpec((_TILE, 2), lambda i: (i, 0)),
        ],
        out_specs=pl.BlockSpec((_TILE, _D), lambda i: (i, 0)),
        out_shape=jax.ShapeDtypeStruct((_N, _D), f32),
        compiler_params=pltpu.CompilerParams(allow_input_fusion=[False, True, False, False, False]),
    )(h, w1t, w2f, eid, gate)

    gate = gate + (jnp.asarray(k, gate.dtype) - 2.0)
    return out, eid, gate, gidx


# speedup vs baseline: 1.0036x; 1.0026x over previous
import jax
import jax.numpy as jnp
from jax import lax
from jax.experimental import pallas as pl
from jax.experimental.pallas import tpu as pltpu
from jax.experimental.pallas import tpu_sc as plsc

_N, _D, _E, _R, _M, _G = 2048, 1024, 64, 16, 8, 8
_TILE = 512
_NEG = -1e30

_NC, _NS, _L = 2, 16, 16
_NW = _NC * _NS
_TPW = _N // _NW


def _score_body(h_ref, wcat_ref, bias_ref, s_ref):
    s = jnp.dot(h_ref[...], wcat_ref[...], preferred_element_type=jnp.float32)
    s_ref[...] = s + bias_ref[0:1, :]


def _sc_route_body(scores_hbm, eid_hbm, gate_hbm, gidx_hbm,
                   sc_v, eid_v, gate_v, gidx_v):
    wid = lax.axis_index("s") * _NC + lax.axis_index("c")
    base = wid * _TPW
    pltpu.sync_copy(scores_hbm.at[pl.ds(base * 128, _TPW * 128)], sc_v)
    lanes = lax.iota(jnp.int32, _L)
    zero = jnp.zeros((_L,), jnp.int32)
    for b in range(_TPW // _L):
        toks = lanes + (b * _L)
        rowoff = toks * 128
        bv = jnp.full((_L,), _NEG, jnp.float32)
        bi = zero
        for j in range(_G):
            v = plsc.load_gather(sc_v, [rowoff + j])
            upd = v > bv
            bv = jnp.where(upd, v, bv)
            bi = jnp.where(upd, zero + j, bi)
        v1 = jnp.full((_L,), _NEG, jnp.float32)
        v2 = jnp.full((_L,), _NEG, jnp.float32)
        i1 = zero
        i2 = zero
        for m in range(_M):
            v = plsc.load_gather(sc_v, [rowoff + (bi * _M + (_G + m))])
            gt1 = v > v1
            gt2 = v > v2
            mv = zero + m
            nv2 = jnp.where(gt1, v1, jnp.where(gt2, v, v2))
            ni2 = jnp.where(gt1, i1, jnp.where(gt2, mv, i2))
            v1 = jnp.where(gt1, v, v1)
            i1 = jnp.where(gt1, mv, i1)
            v2 = nv2
            i2 = ni2
        t = jnp.exp(v2 - v1)
        g2 = t / (1.0 + t)
        g1 = 1.0 - g2
        plsc.store_scatter(eid_v, [toks * 2], bi * _M + i1)
        plsc.store_scatter(eid_v, [toks * 2 + 1], bi * _M + i2)
        plsc.store_scatter(gate_v, [toks * 2], g1)
        plsc.store_scatter(gate_v, [toks * 2 + 1], g2)
        gidx_v[pl.ds(b * _L, _L)] = bi
    pltpu.sync_copy(eid_v, eid_hbm.at[pl.ds(base * 2, _TPW * 2)])
    pltpu.sync_copy(gate_v, gate_hbm.at[pl.ds(base * 2, _TPW * 2)])
    pltpu.sync_copy(gidx_v, gidx_hbm.at[pl.ds(base, _TPW)])


def _expert_body(h_ref, w1_ref, w2_ref, eid_ref, gate_ref, out_ref):
    tmp = jnp.dot(h_ref[...], w1_ref[...], preferred_element_type=jnp.float32)
    tmp = jnp.maximum(tmp, 0.0)
    e1 = eid_ref[:, 0:1]
    e2 = eid_ref[:, 1:2]
    g1 = gate_ref[:, 0:1]
    g2 = gate_ref[:, 1:2]
    elane = lax.broadcasted_iota(jnp.int32, tmp.shape, 1) // _R
    wexp = jnp.where(elane == e1, g1, 0.0) + jnp.where(elane == e2, g2, 0.0)
    out_ref[...] = jnp.dot(tmp * wexp, w2_ref[...],
                           preferred_element_type=jnp.float32)


def kernel(h, k, Wg, bg, local_router, W1, W2):
    f32 = jnp.float32
    wcat = jnp.zeros((_D, 128), f32)
    wcat = wcat.at[:, :_G].set(Wg.T)
    wcat = wcat.at[:, _G:_G + _G * _M].set(
        local_router.transpose(1, 0, 2).reshape(_D, _G * _M))
    bias = jnp.zeros((8, 128), f32).at[0, :_G].set(bg)
    w1t = W1.transpose(1, 0, 2).reshape(_D, _E * _R)
    w2f = W2.reshape(_E * _R, _D)

    grid = _N // _TILE
    scores = pl.pallas_call(
        _score_body,
        grid=(grid,),
        in_specs=[
            pl.BlockSpec((_TILE, _D), lambda i: (i, 0)),
            pl.BlockSpec((_D, 128), lambda i: (0, 0)),
            pl.BlockSpec((8, 128), lambda i: (0, 0)),
        ],
        out_specs=pl.BlockSpec((_TILE, 128), lambda i: (i, 0)),
        out_shape=jax.ShapeDtypeStruct((_N, 128), f32),
    )(h, wcat, bias)

    sc_route = pl.kernel(
        _sc_route_body,
        out_type=[
            jax.ShapeDtypeStruct((_N * 2,), jnp.int32),
            jax.ShapeDtypeStruct((_N * 2,), f32),
            jax.ShapeDtypeStruct((_N,), jnp.int32),
        ],
        mesh=plsc.VectorSubcoreMesh(core_axis_name="c", subcore_axis_name="s",
                                    num_cores=_NC, num_subcores=_NS),
        scratch_types=[
            pltpu.VMEM((_TPW * 128,), f32),
            pltpu.VMEM((_TPW * 2,), jnp.int32),
            pltpu.VMEM((_TPW * 2,), f32),
            pltpu.VMEM((_TPW,), jnp.int32),
        ],
        compiler_params=pltpu.CompilerParams(needs_layout_passes=False),
    )
    eid_flat, gate_flat, gidx = sc_route(scores.reshape(_N * 128))
    eid = eid_flat.reshape(_N, 2)
    gate = gate_flat.reshape(_N, 2)

    out = pl.pallas_call(
        _expert_body,
        grid=(grid,),
        in_specs=[
            pl.BlockSpec((_TILE, _D), lambda i: (i, 0)),
            pl.BlockSpec((_D, _E * _R), lambda i: (0, 0)),
            pl.BlockSpec((_E * _R, _D), lambda i: (0, 0)),
            pl.BlockSpec((_TILE, 2), lambda i: (i, 0)),
            pl.BlockSpec((_TILE, 2), lambda i: (i, 0)),
        ],
        out_specs=pl.BlockSpec((_TILE, _D), lambda i: (i, 0)),
        out_shape=jax.ShapeDtypeStruct((_N, _D), f32),
    )(h, w1t, w2f, eid, gate)

    gate = gate + (jnp.asarray(k, gate.dtype) - 2.0)
    return out, eid, gate, gidx
